# baseline (speedup 1.0000x reference)
import jax
import jax.numpy as jnp
from jax import lax
from jax.experimental import pallas as pl
from jax.experimental.pallas import tpu as pltpu
from jax.experimental.pallas import tpu_sc as plsc

VOCAB_N = 1000000
EMBED_N = 64
BATCH_N = 4096
X_MAX_C = 100.0
ALPHA_C = 0.75

_NUM_WORKERS = 32
_BPW = BATCH_N // _NUM_WORKERS

_ROW_BLOCK = 256


def _sc_gather_body(f_idx, c_idx, fe_hbm, ce_hbm, fb_hbm, cb_hbm,
                    fe_out, ce_out, fb_out, cb_out,
                    fi_v, ci_v, fe_v, ce_v, fb_v, cb_v, sem):
    wid = lax.axis_index("s") * 2 + lax.axis_index("c")
    base = wid * _BPW
    pltpu.sync_copy(f_idx.at[pl.ds(base, _BPW)], fi_v)
    pltpu.sync_copy(c_idx.at[pl.ds(base, _BPW)], ci_v)
    copies = [
        pltpu.async_copy(fe_hbm.at[fi_v], fe_v, sem),
        pltpu.async_copy(ce_hbm.at[ci_v], ce_v, sem),
        pltpu.async_copy(fb_hbm.at[fi_v], fb_v, sem),
        pltpu.async_copy(cb_hbm.at[ci_v], cb_v, sem),
    ]
    for cp in copies:
        cp.wait()
    pltpu.sync_copy(fe_v, fe_out.at[pl.ds(base, _BPW)])
    pltpu.sync_copy(ce_v, ce_out.at[pl.ds(base, _BPW)])
    pltpu.sync_copy(fb_v, fb_out.at[pl.ds(base, _BPW)])
    pltpu.sync_copy(cb_v, cb_out.at[pl.ds(base, _BPW)])


def _dot_body(fe_ref, ce_ref, fb_ref, cb_ref, p_ref, b_ref):
    p_ref[...] = jnp.sum(fe_ref[...] * ce_ref[...], axis=1, keepdims=True)
    b_ref[...] = fb_ref[...] + cb_ref[...]


def _loss_body(p_ref, c_ref, b_ref, o_ref):
    c = c_ref[...]
    a = p_ref[...] - jnp.log(c)
    w = jnp.minimum(jnp.exp(ALPHA_C * jnp.log(c * (1.0 / X_MAX_C))), 1.0)
    s = a + b_ref[...]
    o_ref[...] = w * (s * s)


def kernel(focal_input, context_input, cooccurance_count,
           focal_embedding, context_embedding, focal_biases, context_biases):
    fi = focal_input.astype(jnp.int32)
    ci = context_input.astype(jnp.int32)
    cooc = cooccurance_count.astype(jnp.float32)

    sc_gather = pl.kernel(
        _sc_gather_body,
        out_type=(
            jax.ShapeDtypeStruct((BATCH_N, EMBED_N), jnp.float32),
            jax.ShapeDtypeStruct((BATCH_N, EMBED_N), jnp.float32),
            jax.ShapeDtypeStruct((BATCH_N, 1), jnp.float32),
            jax.ShapeDtypeStruct((BATCH_N, 1), jnp.float32),
        ),
        mesh=plsc.VectorSubcoreMesh(core_axis_name="c", subcore_axis_name="s"),
        compiler_params=pltpu.CompilerParams(use_tc_tiling_on_sc=False),
        scratch_types=[
            pltpu.VMEM((_BPW,), jnp.int32),
            pltpu.VMEM((_BPW,), jnp.int32),
            pltpu.VMEM((_BPW, EMBED_N), jnp.float32),
            pltpu.VMEM((_BPW, EMBED_N), jnp.float32),
            pltpu.VMEM((_BPW, 1), jnp.float32),
            pltpu.VMEM((_BPW, 1), jnp.float32),
            pltpu.SemaphoreType.DMA,
        ],
    )
    fe_g, ce_g, fb_g, cb_g = sc_gather(
        fi, ci, focal_embedding, context_embedding,
        focal_biases, context_biases)

    p, b = pl.pallas_call(
        _dot_body,
        out_shape=(
            jax.ShapeDtypeStruct((BATCH_N, 1), jnp.float32),
            jax.ShapeDtypeStruct((BATCH_N, 1), jnp.float32),
        ),
    )(fe_g, ce_g, fb_g, cb_g)

    out = pl.pallas_call(
        _loss_body,
        grid=(BATCH_N // _ROW_BLOCK,),
        in_specs=[
            pl.BlockSpec((1, BATCH_N), lambda i: (0, 0)),
            pl.BlockSpec((1, BATCH_N), lambda i: (0, 0)),
            pl.BlockSpec((_ROW_BLOCK, 1), lambda i: (i, 0)),
        ],
        out_specs=pl.BlockSpec((_ROW_BLOCK, BATCH_N), lambda i: (i, 0)),
        out_shape=jax.ShapeDtypeStruct((BATCH_N, BATCH_N), jnp.float32),
    )(p.reshape(1, BATCH_N), cooc.reshape(1, BATCH_N), b)
    return out


# speedup vs baseline: 19.1790x; 19.1790x over previous
import jax
import jax.numpy as jnp
from jax import lax
from jax.experimental import pallas as pl
from jax.experimental.pallas import tpu as pltpu
from jax.experimental.pallas import tpu_sc as plsc

VOCAB_N = 1000000
EMBED_N = 64
BATCH_N = 4096
X_MAX_C = 100.0
ALPHA_C = 0.75

_NUM_WORKERS = 32
_BPW = BATCH_N // _NUM_WORKERS
_LANES = 16
_NSLOT = 4
_NITER = _BPW // _NSLOT

_ROW_BLOCK = 512


def _sc_body(f_idx, c_idx, fe_t, ce_t, fb_t, cb_t,
             p_out, b_out,
             fi_v, ci_v,
             feb0, feb1, feb2, feb3, ceb0, ceb1, ceb2, ceb3,
             fbb0, fbb1, fbb2, fbb3, cbb0, cbb1, cbb2, cbb3,
             p_v, b_v, sem0, sem1, sem2, sem3):
    febs = (feb0, feb1, feb2, feb3)
    cebs = (ceb0, ceb1, ceb2, ceb3)
    fbbs = (fbb0, fbb1, fbb2, fbb3)
    cbbs = (cbb0, cbb1, cbb2, cbb3)
    sems = (sem0, sem1, sem2, sem3)

    wid = lax.axis_index("s") * 2 + lax.axis_index("c")
    base = wid * _BPW
    pltpu.sync_copy(f_idx.at[pl.ds(base, _BPW)], fi_v.at[pl.ds(0, _BPW)])
    pltpu.sync_copy(c_idx.at[pl.ds(base, _BPW)], ci_v.at[pl.ds(0, _BPW)])

    lanes = lax.iota(jnp.int32, _LANES)

    def fire(s, jf, jc):
        jtf = pl.multiple_of((jf // 128) * 128, 128)
        jtc = pl.multiple_of((jc // 128) * 128, 128)
        pltpu.async_copy(fe_t.at[:, pl.ds(jtf, 128)], febs[s], sems[s])
        pltpu.async_copy(ce_t.at[:, pl.ds(jtc, 128)], cebs[s], sems[s])
        pltpu.async_copy(fb_t.at[0, pl.ds(jtf, 128)], fbbs[s], sems[s])
        pltpu.async_copy(cb_t.at[0, pl.ds(jtc, 128)], cbbs[s], sems[s])

    def wait_slot(s):
        pltpu.make_async_copy(fe_t.at[:, pl.ds(0, 128)], febs[s], sems[s]).wait()
        pltpu.make_async_copy(ce_t.at[:, pl.ds(0, 128)], cebs[s], sems[s]).wait()
        pltpu.make_async_copy(fb_t.at[0, pl.ds(0, 128)], fbbs[s], sems[s]).wait()
        pltpu.make_async_copy(cb_t.at[0, pl.ds(0, 128)], cbbs[s], sems[s]).wait()

    win_f0 = fi_v[pl.ds(0, _LANES)]
    win_c0 = ci_v[pl.ds(0, _LANES)]
    for s in range(_NSLOT):
        fire(s, win_f0[s], win_c0[s])

    def step(t, carry):
        pvec, bvec = carry
        win_f = fi_v[pl.ds(t * _NSLOT, _LANES)]
        win_c = ci_v[pl.ds(t * _NSLOT, _LANES)]
        win_fn = fi_v[pl.ds(t * _NSLOT + _NSLOT, _LANES)]
        win_cn = ci_v[pl.ds(t * _NSLOT + _NSLOT, _LANES)]
        lbase = (t % 4) * _NSLOT
        for s in range(_NSLOT):
            wait_slot(s)
            jl_f = win_f[s] % 128
            jl_c = win_c[s] % 128
            colf = jnp.zeros((_LANES,), jnp.int32) + jl_f
            colc = jnp.zeros((_LANES,), jnp.int32) + jl_c
            acc = jnp.zeros((_LANES,), jnp.float32)
            for g in range(EMBED_N // _LANES):
                rows = lanes + (g * _LANES)
                fv = plsc.load_gather(febs[s], [rows, colf])
                cv = plsc.load_gather(cebs[s], [rows, colc])
                acc = acc + fv * cv
            p_s = jnp.sum(acc)
            fbv = plsc.load_gather(fbbs[s], [colf])
            cbv = plsc.load_gather(cbbs[s], [colc])
            b_s = fbv[0] + cbv[0]

            @pl.when(t < _NITER - 1)
            def _():
                fire(s, win_fn[s], win_cn[s])

            msk = lanes == (lbase + s)
            pvec = jnp.where(msk, p_s, pvec)
            bvec = jnp.where(msk, b_s, bvec)

        @pl.when(t % 4 == 3)
        def _():
            p_v[pl.ds((t // 4) * _LANES, _LANES)] = pvec
            b_v[pl.ds((t // 4) * _LANES, _LANES)] = bvec

        done = (jnp.zeros((_LANES,), jnp.int32) + (t % 4)) == 3
        pvec = jnp.where(done, 0.0, pvec)
        bvec = jnp.where(done, 0.0, bvec)
        return pvec, bvec

    lax.fori_loop(0, _NITER, step,
                  (jnp.zeros((_LANES,), jnp.float32),
                   jnp.zeros((_LANES,), jnp.float32)))

    pltpu.sync_copy(p_v, p_out.at[pl.ds(base, _BPW)])
    pltpu.sync_copy(b_v, b_out.at[pl.ds(base, _BPW)])


def _loss_body(p_ref, c_ref, b_ref, o_ref):
    c = c_ref[...]
    a = p_ref[...] - jnp.log(c)
    w = jnp.minimum(jnp.exp(ALPHA_C * jnp.log(c * (1.0 / X_MAX_C))), 1.0)
    s = a + b_ref[...]
    o_ref[...] = w * (s * s)


def kernel(focal_input, context_input, cooccurance_count,
           focal_embedding, context_embedding, focal_biases, context_biases):
    fi = focal_input.astype(jnp.int32)
    ci = context_input.astype(jnp.int32)
    cooc = cooccurance_count.astype(jnp.float32)

    emb_buf = pltpu.VMEM((EMBED_N, 128), jnp.float32)
    bias_buf = pltpu.VMEM((128,), jnp.float32)
    sc_gather = pl.kernel(
        _sc_body,
        out_type=(
            jax.ShapeDtypeStruct((BATCH_N,), jnp.float32),
            jax.ShapeDtypeStruct((BATCH_N,), jnp.float32),
        ),
        mesh=plsc.VectorSubcoreMesh(core_axis_name="c", subcore_axis_name="s"),
        compiler_params=pltpu.CompilerParams(needs_layout_passes=False),
        scratch_types=(
            [pltpu.VMEM((_BPW + _LANES,), jnp.int32)] * 2
            + [emb_buf] * 8
            + [bias_buf] * 8
            + [pltpu.VMEM((_BPW,), jnp.float32)] * 2
            + [pltpu.SemaphoreType.DMA] * 4
        ),
    )
    p, b = sc_gather(fi, ci, focal_embedding.T, context_embedding.T,
                     focal_biases.T, context_biases.T)

    out = pl.pallas_call(
        _loss_body,
        grid=(BATCH_N // _ROW_BLOCK,),
        in_specs=[
            pl.BlockSpec((1, BATCH_N), lambda i: (0, 0)),
            pl.BlockSpec((1, BATCH_N), lambda i: (0, 0)),
            pl.BlockSpec((_ROW_BLOCK, 1), lambda i: (i, 0)),
        ],
        out_specs=pl.BlockSpec((_ROW_BLOCK, BATCH_N), lambda i: (i, 0)),
        out_shape=jax.ShapeDtypeStruct((BATCH_N, BATCH_N), jnp.float32),
    )(p.reshape(1, BATCH_N), cooc.reshape(1, BATCH_N), b.reshape(BATCH_N, 1))
    return out
